# baseline (speedup 1.0000x reference)
import math

import jax
import jax.numpy as jnp
from jax import lax
from jax.experimental import pallas as pl
from jax.experimental.pallas import tpu as pltpu
from jax.experimental.pallas import tpu_sc as plsc

NC = 2
NS = 16
NW = NC * NS
EDGE_CHUNK = 88
CPT = 114
NROWS = 4
PAD_ROWS = 16
BN_SCALE = 1.0 / math.sqrt(1.0 + 1e-5)


def _scatter_body(z_hbm, src_hbm, dst_hbm, zero_hbm, out_hbm,
                  sb0, sb1, sb2, sb3, db0, db1, db2, db3, db4, db5,
                  rows, acc,
                  gs0, gs1, gs2, ss0, ss1,
                  iss0, iss1, iss2, iss3, isd0, isd1, isd2, isd3, isd4, isd5):
    sbufs = (sb0, sb1, sb2, sb3)
    dbufs = (db0, db1, db2, db3, db4, db5)
    gsems, ssems = (gs0, gs1, gs2), (ss0, ss1)
    isems_s = (iss0, iss1, iss2, iss3)
    isems_d = (isd0, isd1, isd2, isd3, isd4, isd5)
    cid = lax.axis_index("c")
    sid = lax.axis_index("s")
    n = z_hbm.shape[0]
    wid = cid * NS + sid
    c0 = wid * CPT

    def idx_start(j, u):
        off = (c0 + j) * EDGE_CHUNK
        pltpu.async_copy(src_hbm.at[pl.ds(off, EDGE_CHUNK)], sbufs[u % 4],
                         isems_s[u % 4])
        pltpu.async_copy(dst_hbm.at[pl.ds(off, EDGE_CHUNK)], dbufs[u % 6],
                         isems_d[u % 6])

    def idx_wait(j, u):
        off = (c0 + j) * EDGE_CHUNK
        pltpu.make_async_copy(src_hbm.at[pl.ds(off, EDGE_CHUNK)],
                              sbufs[u % 4], isems_s[u % 4]).wait()
        pltpu.make_async_copy(dst_hbm.at[pl.ds(off, EDGE_CHUNK)],
                              dbufs[u % 6], isems_d[u % 6]).wait()

    def gather_start(u):
        pltpu.async_copy(z_hbm.at[sbufs[u % 4]], rows.at[u % 4],
                         gsems[u % 3])

    def gather_wait(u):
        pltpu.make_async_copy(z_hbm.at[sbufs[u % 4]], rows.at[u % 4],
                              gsems[u % 3]).wait()

    def scat_start(u):
        pltpu.async_copy(rows.at[u % 4], acc.at[dbufs[u % 6]], ssems[u % 2],
                         add=True)

    def scat_wait(u):
        pltpu.make_async_copy(rows.at[u % 4], acc.at[dbufs[u % 6]],
                              ssems[u % 2]).wait()

    for jj in range(4):
        idx_start(jj, jj)

    cp = ((n // NS + 7) // 8) * 8
    tail = n - cp * (NS - 1)
    r0 = sid * cp

    def _init_range(start, length):
        pltpu.sync_copy(zero_hbm.at[pl.ds(start, length)],
                        acc.at[pl.ds(start, length)])

    @pl.when(sid < NS - 1)
    def _():
        _init_range(r0, cp)

    @pl.when(sid == NS - 1)
    def _():
        _init_range(cp * (NS - 1), tail)

    idx_wait(0, 0)
    gather_start(0)
    idx_wait(1, 1)
    gather_start(1)
    plsc.subcore_barrier()

    def step(j, u, scat_w, nxt, idx_st):
        if scat_w:
            scat_wait(u - 2)
        if nxt:
            idx_wait(j + 2, u + 2)
            gather_start(u + 2)
        gather_wait(u)
        scat_start(u)
        if idx_st:
            idx_start(j + 4, u + 4)

    step(0, 0, scat_w=False, nxt=True, idx_st=True)
    step(1, 1, scat_w=False, nxt=True, idx_st=True)

    def body(k, carry):
        base = 2 + 12 * k
        for t in range(12):
            step(base + t, 2 + t, scat_w=True, nxt=True, idx_st=True)
        return carry

    lax.fori_loop(0, (CPT - 6) // 12, body, 0)

    for jj in range(CPT - 4, CPT):
        step(jj, jj, scat_w=True, nxt=(jj + 2 < CPT), idx_st=False)
    scat_wait(CPT - 2)
    scat_wait(CPT - 1)

    plsc.subcore_barrier()

    def _pub_range(start, length):
        pltpu.sync_copy(acc.at[pl.ds(start, length)],
                        out_hbm.at[pl.ds(cid * n + start, length)])

    @pl.when(sid < NS - 1)
    def _():
        _pub_range(r0, cp)

    @pl.when(sid == NS - 1)
    def _():
        _pub_range(cp * (NS - 1), tail)


def _make_scatter(n, d, dtype):
    mesh = plsc.VectorSubcoreMesh(core_axis_name="c", subcore_axis_name="s",
                                  num_cores=NC, num_subcores=NS)
    return pl.kernel(
        _scatter_body,
        out_type=jax.ShapeDtypeStruct((NC * n, d), dtype),
        mesh=mesh,
        scratch_types=(
            [pltpu.VMEM((EDGE_CHUNK,), jnp.int32) for _ in range(10)]
            + [pltpu.VMEM((NROWS, EDGE_CHUNK, d), dtype),
               pltpu.VMEM_SHARED((n + PAD_ROWS, d), dtype)]
            + [pltpu.SemaphoreType.DMA for _ in range(15)]
        ),
    )


def _mlp_body(x_ref, p0_ref, p1_ref, wa_ref, ba_ref, wb_ref, bb_ref, o_ref):
    h = x_ref[...] + p0_ref[...] + p1_ref[...]
    h = jnp.maximum(
        jnp.dot(h, wa_ref[...], preferred_element_type=jnp.float32)
        + ba_ref[...], 0.0)
    h = jnp.dot(h, wb_ref[...], preferred_element_type=jnp.float32) + bb_ref[...]
    o_ref[...] = jnp.maximum(h, 0.0)


def _tail_body(x_ref, p0_ref, p1_ref, wa_ref, ba_ref, wb_ref, bb_ref,
               bng_ref, bnb_ref, wp_ref, bp_ref, bn2g_ref, bn2b_ref, pa_ref,
               z_ref, p_ref):
    h = x_ref[...] + p0_ref[...] + p1_ref[...]
    h = jnp.maximum(
        jnp.dot(h, wa_ref[...], preferred_element_type=jnp.float32)
        + ba_ref[...], 0.0)
    h = jnp.dot(h, wb_ref[...], preferred_element_type=jnp.float32) + bb_ref[...]
    h = jnp.maximum(h, 0.0)
    z = h * (bng_ref[...] * BN_SCALE) + bnb_ref[...]
    z_ref[...] = z
    q = jnp.dot(z, wp_ref[...], preferred_element_type=jnp.float32) + bp_ref[...]
    q = q * (bn2g_ref[...] * BN_SCALE) + bn2b_ref[...]
    p_ref[...] = jnp.where(q > 0, q, pa_ref[0, 0] * q)


_BLK = 2000


def _row_spec(d):
    return pl.BlockSpec((_BLK, d), lambda i: (i, 0))


def _rep_spec(r, c):
    return pl.BlockSpec((r, c), lambda i: (0, 0))


def _mlp_call(x, p0, p1, wa, ba, wb, bb):
    n, d = x.shape
    h = wa.shape[1]
    return pl.pallas_call(
        _mlp_body,
        grid=(n // _BLK,),
        in_specs=[_row_spec(d), _row_spec(d), _row_spec(d),
                  _rep_spec(d, h), _rep_spec(1, h),
                  _rep_spec(h, h), _rep_spec(1, h)],
        out_specs=_row_spec(h),
        out_shape=jax.ShapeDtypeStruct((n, h), x.dtype),
    )(x, p0, p1, wa, ba.reshape(1, h), wb, bb.reshape(1, h))


def _tail_call(x, p0, p1, wa, ba, wb, bb, bng, bnb, wp, bp, bn2g, bn2b, pa):
    n, d = x.shape
    h = wa.shape[1]
    vecs = [a.reshape(1, h) for a in (ba, bb, bng, bnb, bp, bn2g, bn2b)]
    return pl.pallas_call(
        _tail_body,
        grid=(n // _BLK,),
        in_specs=[_row_spec(d), _row_spec(d), _row_spec(d),
                  _rep_spec(d, h), _rep_spec(1, h),
                  _rep_spec(h, h), _rep_spec(1, h),
                  _rep_spec(1, h), _rep_spec(1, h),
                  _rep_spec(h, h), _rep_spec(1, h),
                  _rep_spec(1, h), _rep_spec(1, h),
                  pl.BlockSpec(memory_space=pltpu.SMEM)],
        out_specs=[_row_spec(h), _row_spec(h)],
        out_shape=[jax.ShapeDtypeStruct((n, h), x.dtype),
                   jax.ShapeDtypeStruct((n, h), x.dtype)],
    )(x, p0, p1, wa, vecs[0], wb, vecs[1], vecs[2], vecs[3], wp, vecs[4],
      vecs[5], vecs[6], pa.reshape(1, 1))


def kernel(x, edge_index, W1a, b1a, W1b, b1b, W2a, b2a, W2b, b2b,
           bn_gamma, bn_beta, Wp, bp, bn2_gamma, bn2_beta, prelu_a):
    n, d = x.shape
    e = edge_index.shape[1]
    e_pad = NW * CPT * EDGE_CHUNK
    ar = jnp.arange(e_pad - e, dtype=jnp.int32)
    src = jnp.concatenate([edge_index[0], ar % n])
    dst = jnp.concatenate([edge_index[1], n + (ar % PAD_ROWS)])
    zeros = jnp.zeros_like(x)
    scat = _make_scatter(n, d, x.dtype)

    agg1 = scat(x, src, dst, zeros)
    z1 = _mlp_call(x, agg1[:n], agg1[n:], W1a, b1a, W1b, b1b)
    agg2 = scat(z1, src, dst, zeros)
    z, p = _tail_call(z1, agg2[:n], agg2[n:], W2a, b2a, W2b, b2b,
                      bn_gamma, bn_beta, Wp, bp, bn2_gamma, bn2_beta, prelu_a)
    return (z, p)


# speedup vs baseline: 1.0103x; 1.0103x over previous
import math

import jax
import jax.numpy as jnp
from jax import lax
from jax.experimental import pallas as pl
from jax.experimental.pallas import tpu as pltpu
from jax.experimental.pallas import tpu_sc as plsc

NC = 2
NS = 16
NW = NC * NS
EDGE_CHUNK = 128
CPT = 80
NBUF = 3
PAD_ROWS = 16
BN_SCALE = 1.0 / math.sqrt(1.0 + 1e-5)


def _scatter_body(z_hbm, src_hbm, dst_hbm, zero_hbm, out_hbm,
                  sb0, sb1, sb2, db0, db1, db2, db3, rows, acc,
                  gs0, gs1, ss0, ss1, ss2, ss3,
                  iss0, iss1, iss2, isd0, isd1, isd2, isd3):
    sbufs, dbufs = (sb0, sb1, sb2), (db0, db1, db2, db3)
    gsems, ssems = (gs0, gs1), (ss0, ss1, ss2, ss3)
    isems_s, isems_d = (iss0, iss1, iss2), (isd0, isd1, isd2, isd3)
    cid = lax.axis_index("c")
    sid = lax.axis_index("s")
    n = z_hbm.shape[0]
    wid = cid * NS + sid
    c0 = wid * CPT

    def idx_start(j, s3, s4):
        off = (c0 + j) * EDGE_CHUNK
        pltpu.async_copy(src_hbm.at[pl.ds(off, EDGE_CHUNK)], sbufs[s3],
                         isems_s[s3])
        pltpu.async_copy(dst_hbm.at[pl.ds(off, EDGE_CHUNK)], dbufs[s4],
                         isems_d[s4])

    def idx_wait(j, s3, s4):
        off = (c0 + j) * EDGE_CHUNK
        pltpu.make_async_copy(src_hbm.at[pl.ds(off, EDGE_CHUNK)], sbufs[s3],
                              isems_s[s3]).wait()
        pltpu.make_async_copy(dst_hbm.at[pl.ds(off, EDGE_CHUNK)], dbufs[s4],
                              isems_d[s4]).wait()

    def gather_start(s3, g2):
        pltpu.async_copy(z_hbm.at[sbufs[s3]], rows.at[s3], gsems[g2])

    def gather_wait(s3, g2):
        pltpu.make_async_copy(z_hbm.at[sbufs[s3]], rows.at[s3],
                              gsems[g2]).wait()

    def scat_start(s3, s4):
        pltpu.async_copy(rows.at[s3], acc.at[dbufs[s4]], ssems[s4], add=True)

    def scat_wait(s3, s4):
        pltpu.make_async_copy(rows.at[s3], acc.at[dbufs[s4]],
                              ssems[s4]).wait()

    idx_start(0, 0, 0)
    idx_start(1, 1, 1)

    cp = ((n // NS + 7) // 8) * 8
    tail = n - cp * (NS - 1)
    r0 = sid * cp

    def _init_range(start, length):
        pltpu.sync_copy(zero_hbm.at[pl.ds(start, length)],
                        acc.at[pl.ds(start, length)])

    @pl.when(sid < NS - 1)
    def _():
        _init_range(r0, cp)

    @pl.when(sid == NS - 1)
    def _():
        _init_range(cp * (NS - 1), tail)

    idx_wait(0, 0, 0)
    gather_start(0, 0)
    plsc.subcore_barrier()

    def step(j, s3, s4, g2, scat_w, idx_st, nxt):
        if scat_w:
            scat_wait((s3 + 1) % 3, (s4 + 2) % 4)
        if nxt:
            idx_wait(j + 1, (s3 + 1) % 3, (s4 + 1) % 4)
            gather_start((s3 + 1) % 3, (g2 + 1) % 2)
        if idx_st:
            idx_start(j + 2, (s3 + 2) % 3, (s4 + 2) % 4)
        gather_wait(s3, g2)
        scat_start(s3, s4)

    for jj in range(4):
        step(jj, jj % 3, jj % 4, jj % 2,
             scat_w=(jj >= 2), idx_st=True, nxt=True)

    def body(k, carry):
        base = 4 + 12 * k
        for t in range(12):
            step(base + t, (4 + t) % 3, t % 4, t % 2,
                 scat_w=True, idx_st=True, nxt=True)
        return carry

    lax.fori_loop(0, (CPT - 8) // 12, body, 0)

    for jj in range(CPT - 4, CPT):
        step(jj, jj % 3, jj % 4, jj % 2,
             scat_w=True, idx_st=(jj + 2 < CPT), nxt=(jj + 1 < CPT))
    scat_wait((CPT - 2) % 3, (CPT - 2) % 4)
    scat_wait((CPT - 1) % 3, (CPT - 1) % 4)

    plsc.subcore_barrier()

    def _pub_range(start, length):
        pltpu.sync_copy(acc.at[pl.ds(start, length)],
                        out_hbm.at[pl.ds(cid * n + start, length)])

    @pl.when(sid < NS - 1)
    def _():
        _pub_range(r0, cp)

    @pl.when(sid == NS - 1)
    def _():
        _pub_range(cp * (NS - 1), tail)


def _make_scatter(n, d, dtype):
    mesh = plsc.VectorSubcoreMesh(core_axis_name="c", subcore_axis_name="s",
                                  num_cores=NC, num_subcores=NS)
    return pl.kernel(
        _scatter_body,
        out_type=jax.ShapeDtypeStruct((NC * n, d), dtype),
        mesh=mesh,
        scratch_types=(
            [pltpu.VMEM((EDGE_CHUNK,), jnp.int32) for _ in range(7)]
            + [pltpu.VMEM((NBUF, EDGE_CHUNK, d), dtype),
               pltpu.VMEM_SHARED((n + PAD_ROWS, d), dtype)]
            + [pltpu.SemaphoreType.DMA for _ in range(13)]
        ),
    )


def _mlp_body(x_ref, p0_ref, p1_ref, wa_ref, ba_ref, wb_ref, bb_ref, o_ref):
    h = x_ref[...] + p0_ref[...] + p1_ref[...]
    h = jnp.maximum(
        jnp.dot(h, wa_ref[...], preferred_element_type=jnp.float32)
        + ba_ref[...], 0.0)
    h = jnp.dot(h, wb_ref[...], preferred_element_type=jnp.float32) + bb_ref[...]
    o_ref[...] = jnp.maximum(h, 0.0)


def _tail_body(x_ref, p0_ref, p1_ref, wa_ref, ba_ref, wb_ref, bb_ref,
               bng_ref, bnb_ref, wp_ref, bp_ref, bn2g_ref, bn2b_ref, pa_ref,
               z_ref, p_ref):
    h = x_ref[...] + p0_ref[...] + p1_ref[...]
    h = jnp.maximum(
        jnp.dot(h, wa_ref[...], preferred_element_type=jnp.float32)
        + ba_ref[...], 0.0)
    h = jnp.dot(h, wb_ref[...], preferred_element_type=jnp.float32) + bb_ref[...]
    h = jnp.maximum(h, 0.0)
    z = h * (bng_ref[...] * BN_SCALE) + bnb_ref[...]
    z_ref[...] = z
    q = jnp.dot(z, wp_ref[...], preferred_element_type=jnp.float32) + bp_ref[...]
    q = q * (bn2g_ref[...] * BN_SCALE) + bn2b_ref[...]
    p_ref[...] = jnp.where(q > 0, q, pa_ref[0, 0] * q)


_BLK = 2000


def _row_spec(d):
    return pl.BlockSpec((_BLK, d), lambda i: (i, 0))


def _rep_spec(r, c):
    return pl.BlockSpec((r, c), lambda i: (0, 0))


def _mlp_call(x, p0, p1, wa, ba, wb, bb):
    n, d = x.shape
    h = wa.shape[1]
    return pl.pallas_call(
        _mlp_body,
        grid=(n // _BLK,),
        in_specs=[_row_spec(d), _row_spec(d), _row_spec(d),
                  _rep_spec(d, h), _rep_spec(1, h),
                  _rep_spec(h, h), _rep_spec(1, h)],
        out_specs=_row_spec(h),
        out_shape=jax.ShapeDtypeStruct((n, h), x.dtype),
    )(x, p0, p1, wa, ba.reshape(1, h), wb, bb.reshape(1, h))


def _tail_call(x, p0, p1, wa, ba, wb, bb, bng, bnb, wp, bp, bn2g, bn2b, pa):
    n, d = x.shape
    h = wa.shape[1]
    vecs = [a.reshape(1, h) for a in (ba, bb, bng, bnb, bp, bn2g, bn2b)]
    return pl.pallas_call(
        _tail_body,
        grid=(n // _BLK,),
        in_specs=[_row_spec(d), _row_spec(d), _row_spec(d),
                  _rep_spec(d, h), _rep_spec(1, h),
                  _rep_spec(h, h), _rep_spec(1, h),
                  _rep_spec(1, h), _rep_spec(1, h),
                  _rep_spec(h, h), _rep_spec(1, h),
                  _rep_spec(1, h), _rep_spec(1, h),
                  pl.BlockSpec(memory_space=pltpu.SMEM)],
        out_specs=[_row_spec(h), _row_spec(h)],
        out_shape=[jax.ShapeDtypeStruct((n, h), x.dtype),
                   jax.ShapeDtypeStruct((n, h), x.dtype)],
    )(x, p0, p1, wa, vecs[0], wb, vecs[1], vecs[2], vecs[3], wp, vecs[4],
      vecs[5], vecs[6], pa.reshape(1, 1))


def kernel(x, edge_index, W1a, b1a, W1b, b1b, W2a, b2a, W2b, b2b,
           bn_gamma, bn_beta, Wp, bp, bn2_gamma, bn2_beta, prelu_a):
    n, d = x.shape
    e = edge_index.shape[1]
    e_pad = NW * CPT * EDGE_CHUNK
    ar = jnp.arange(e_pad - e, dtype=jnp.int32)
    src = jnp.concatenate([edge_index[0], ar % n])
    dst = jnp.concatenate([edge_index[1], n + (ar % PAD_ROWS)])
    zeros = jnp.zeros_like(x)
    scat = _make_scatter(n, d, x.dtype)

    agg1 = scat(x, src, dst, zeros)
    z1 = _mlp_call(x, agg1[:n], agg1[n:], W1a, b1a, W1b, b1b)
    agg2 = scat(z1, src, dst, zeros)
    z, p = _tail_call(z1, agg2[:n], agg2[n:], W2a, b2a, W2b, b2b,
                      bn_gamma, bn_beta, Wp, bp, bn2_gamma, bn2_beta, prelu_a)
    return (z, p)


# speedup vs baseline: 1.0651x; 1.0543x over previous
import math

import jax
import jax.numpy as jnp
from jax import lax
from jax.experimental import pallas as pl
from jax.experimental.pallas import tpu as pltpu
from jax.experimental.pallas import tpu_sc as plsc

NC = 2
NS = 16
NW = NC * NS
EDGE_CHUNK = 128
CPT = 80
NBUF = 3
PAD_ROWS = 16
BN_SCALE = 1.0 / math.sqrt(1.0 + 1e-5)


def _scatter_body(z_hbm, src_hbm, dst_hbm, zero_hbm, out_hbm,
                  sb0, sb1, sb2, db0, db1, db2, db3, rows, acc,
                  gs0, gs1, ss0, ss1, ss2, ss3,
                  iss0, iss1, iss2, isd0, isd1, isd2, isd3):
    sbufs, dbufs = (sb0, sb1, sb2), (db0, db1, db2, db3)
    gsems, ssems = (gs0, gs1), (ss0, ss1, ss2, ss3)
    isems_s, isems_d = (iss0, iss1, iss2), (isd0, isd1, isd2, isd3)
    cid = lax.axis_index("c")
    sid = lax.axis_index("s")
    n = z_hbm.shape[0]
    wid = cid * NS + sid
    c0 = wid * CPT

    def idx_start(j, s3, s4):
        off = (c0 + j) * EDGE_CHUNK
        pltpu.async_copy(src_hbm.at[pl.ds(off, EDGE_CHUNK)], sbufs[s3],
                         isems_s[s3])
        pltpu.async_copy(dst_hbm.at[pl.ds(off, EDGE_CHUNK)], dbufs[s4],
                         isems_d[s4])

    def idx_wait(j, s3, s4):
        off = (c0 + j) * EDGE_CHUNK
        pltpu.make_async_copy(src_hbm.at[pl.ds(off, EDGE_CHUNK)], sbufs[s3],
                              isems_s[s3]).wait()
        pltpu.make_async_copy(dst_hbm.at[pl.ds(off, EDGE_CHUNK)], dbufs[s4],
                              isems_d[s4]).wait()

    def gather_start(s3, g2):
        pltpu.async_copy(z_hbm.at[sbufs[s3]], rows.at[s3], gsems[g2])

    def gather_wait(s3, g2):
        pltpu.make_async_copy(z_hbm.at[sbufs[s3]], rows.at[s3],
                              gsems[g2]).wait()

    def scat_start(s3, s4):
        pltpu.async_copy(rows.at[s3], acc.at[dbufs[s4]], ssems[s4], add=True)

    def scat_wait(s3, s4):
        pltpu.make_async_copy(rows.at[s3], acc.at[dbufs[s4]],
                              ssems[s4]).wait()

    idx_start(0, 0, 0)
    idx_start(1, 1, 1)

    cp = ((n // NS + 7) // 8) * 8
    tail = n - cp * (NS - 1)
    r0 = sid * cp

    def _init_range(start, length):
        pltpu.sync_copy(zero_hbm.at[pl.ds(start, length)],
                        acc.at[pl.ds(start, length)])

    @pl.when(sid < NS - 1)
    def _():
        _init_range(r0, cp)

    @pl.when(sid == NS - 1)
    def _():
        _init_range(cp * (NS - 1), tail)

    idx_wait(0, 0, 0)
    gather_start(0, 0)
    plsc.subcore_barrier()

    def step(j, s3, s4, g2, scat_w, idx_st, nxt):
        if scat_w:
            scat_wait((s3 + 1) % 3, (s4 + 2) % 4)
        if nxt:
            idx_wait(j + 1, (s3 + 1) % 3, (s4 + 1) % 4)
            gather_start((s3 + 1) % 3, (g2 + 1) % 2)
        if idx_st:
            idx_start(j + 2, (s3 + 2) % 3, (s4 + 2) % 4)
        gather_wait(s3, g2)
        scat_start(s3, s4)

    for jj in range(4):
        step(jj, jj % 3, jj % 4, jj % 2,
             scat_w=(jj >= 2), idx_st=True, nxt=True)

    def body(k, carry):
        base = 4 + 12 * k
        for t in range(12):
            step(base + t, (4 + t) % 3, t % 4, t % 2,
                 scat_w=True, idx_st=True, nxt=True)
        return carry

    lax.fori_loop(0, (CPT - 8) // 12, body, 0)

    for jj in range(CPT - 4, CPT):
        step(jj, jj % 3, jj % 4, jj % 2,
             scat_w=True, idx_st=(jj + 2 < CPT), nxt=(jj + 1 < CPT))
    scat_wait((CPT - 2) % 3, (CPT - 2) % 4)
    scat_wait((CPT - 1) % 3, (CPT - 1) % 4)

    plsc.subcore_barrier()

    def _pub_range(start, length):
        pltpu.sync_copy(acc.at[pl.ds(start, length)],
                        out_hbm.at[pl.ds(cid * n + start, length)])

    @pl.when(sid < NS - 1)
    def _():
        _pub_range(r0, cp)

    @pl.when(sid == NS - 1)
    def _():
        _pub_range(cp * (NS - 1), tail)


def _make_scatter(n, d, dtype):
    mesh = plsc.VectorSubcoreMesh(core_axis_name="c", subcore_axis_name="s",
                                  num_cores=NC, num_subcores=NS)
    return pl.kernel(
        _scatter_body,
        out_type=jax.ShapeDtypeStruct((NC * n, d), dtype),
        mesh=mesh,
        scratch_types=(
            [pltpu.VMEM((EDGE_CHUNK,), jnp.int32) for _ in range(7)]
            + [pltpu.VMEM((NBUF, EDGE_CHUNK, d), dtype),
               pltpu.VMEM_SHARED((n + PAD_ROWS, d), dtype)]
            + [pltpu.SemaphoreType.DMA for _ in range(13)]
        ),
    )


def _mlp_body(x_ref, p0_ref, p1_ref, wa_ref, ba_ref, wb_ref, bb_ref, o_ref):
    h = x_ref[...] + p0_ref[...] + p1_ref[...]
    h = jnp.maximum(
        jnp.dot(h, wa_ref[...], preferred_element_type=jnp.float32)
        + ba_ref[...], 0.0)
    h = jnp.dot(h, wb_ref[...], preferred_element_type=jnp.float32) + bb_ref[...]
    o_ref[...] = jnp.maximum(h, 0.0)


def _tail_body(x_ref, p0_ref, p1_ref, wa_ref, ba_ref, wb_ref, bb_ref,
               bng_ref, bnb_ref, wp_ref, bp_ref, bn2g_ref, bn2b_ref, pa_ref,
               z_ref, p_ref):
    h = x_ref[...] + p0_ref[...] + p1_ref[...]
    h = jnp.maximum(
        jnp.dot(h, wa_ref[...], preferred_element_type=jnp.float32)
        + ba_ref[...], 0.0)
    h = jnp.dot(h, wb_ref[...], preferred_element_type=jnp.float32) + bb_ref[...]
    h = jnp.maximum(h, 0.0)
    z = h * (bng_ref[...] * BN_SCALE) + bnb_ref[...]
    z_ref[...] = z
    q = jnp.dot(z, wp_ref[...], preferred_element_type=jnp.float32) + bp_ref[...]
    q = q * (bn2g_ref[...] * BN_SCALE) + bn2b_ref[...]
    p_ref[...] = jnp.where(q > 0, q, pa_ref[0, 0] * q)


_BLK = 2000


def _row_spec(d):
    return pl.BlockSpec((_BLK, d), lambda i: (i, 0))


def _rep_spec(r, c):
    return pl.BlockSpec((r, c), lambda i: (0, 0))


def _half_spec(d, half_blocks):
    return pl.BlockSpec((_BLK, d), lambda i, h=half_blocks: (i + h, 0))


def _mlp_call(x, agg, wa, ba, wb, bb):
    n, d = x.shape
    h = wa.shape[1]
    nb = n // _BLK
    return pl.pallas_call(
        _mlp_body,
        grid=(nb,),
        in_specs=[_row_spec(d), _half_spec(d, 0), _half_spec(d, nb),
                  _rep_spec(d, h), _rep_spec(1, h),
                  _rep_spec(h, h), _rep_spec(1, h)],
        out_specs=_row_spec(h),
        out_shape=jax.ShapeDtypeStruct((n, h), x.dtype),
    )(x, agg, agg, wa, ba.reshape(1, h), wb, bb.reshape(1, h))


def _tail_call(x, agg, wa, ba, wb, bb, bng, bnb, wp, bp, bn2g, bn2b, pa):
    n, d = x.shape
    h = wa.shape[1]
    nb = n // _BLK
    vecs = [a.reshape(1, h) for a in (ba, bb, bng, bnb, bp, bn2g, bn2b)]
    return pl.pallas_call(
        _tail_body,
        grid=(nb,),
        in_specs=[_row_spec(d), _half_spec(d, 0), _half_spec(d, nb),
                  _rep_spec(d, h), _rep_spec(1, h),
                  _rep_spec(h, h), _rep_spec(1, h),
                  _rep_spec(1, h), _rep_spec(1, h),
                  _rep_spec(h, h), _rep_spec(1, h),
                  _rep_spec(1, h), _rep_spec(1, h),
                  pl.BlockSpec(memory_space=pltpu.SMEM)],
        out_specs=[_row_spec(h), _row_spec(h)],
        out_shape=[jax.ShapeDtypeStruct((n, h), x.dtype),
                   jax.ShapeDtypeStruct((n, h), x.dtype)],
    )(x, agg, agg, wa, vecs[0], wb, vecs[1], vecs[2], vecs[3], wp, vecs[4],
      vecs[5], vecs[6], pa.reshape(1, 1))


def kernel(x, edge_index, W1a, b1a, W1b, b1b, W2a, b2a, W2b, b2b,
           bn_gamma, bn_beta, Wp, bp, bn2_gamma, bn2_beta, prelu_a):
    n, d = x.shape
    e = edge_index.shape[1]
    e_pad = NW * CPT * EDGE_CHUNK
    ar = jnp.arange(e_pad - e, dtype=jnp.int32)
    src = jnp.concatenate([edge_index[0], ar % n])
    dst = jnp.concatenate([edge_index[1], n + (ar % PAD_ROWS)])
    zeros = jnp.zeros_like(x)
    scat = _make_scatter(n, d, x.dtype)

    agg1 = scat(x, src, dst, zeros)
    z1 = _mlp_call(x, agg1, W1a, b1a, W1b, b1b)
    agg2 = scat(z1, src, dst, zeros)
    z, p = _tail_call(z1, agg2, W2a, b2a, W2b, b2b,
                      bn_gamma, bn_beta, Wp, bp, bn2_gamma, bn2_beta, prelu_a)
    return (z, p)


# speedup vs baseline: 1.0829x; 1.0168x over previous
import math

import jax
import jax.numpy as jnp
from jax import lax
from jax.experimental import pallas as pl
from jax.experimental.pallas import tpu as pltpu
from jax.experimental.pallas import tpu_sc as plsc

NC = 2
NS = 16
NW = NC * NS
EDGE_CHUNK = 128
CPT = 80
NBUF = 3
PAD_ROWS = 16
BN_SCALE = 1.0 / math.sqrt(1.0 + 1e-5)


def _scatter_body(z_hbm, src_hbm, dst_hbm, zero_hbm, out_hbm,
                  sb0, sb1, sb2, db0, db1, db2, db3, rows, acc,
                  gs0, gs1, ss0, ss1, ss2, ss3,
                  iss0, iss1, iss2, isd0, isd1, isd2, isd3):
    sbufs, dbufs = (sb0, sb1, sb2), (db0, db1, db2, db3)
    gsems, ssems = (gs0, gs1), (ss0, ss1, ss2, ss3)
    isems_s, isems_d = (iss0, iss1, iss2), (isd0, isd1, isd2, isd3)
    cid = lax.axis_index("c")
    sid = lax.axis_index("s")
    n = z_hbm.shape[0]
    wid = cid * NS + sid
    c0 = wid * CPT

    def idx_start(j, s3, s4):
        off = (c0 + j) * EDGE_CHUNK
        pltpu.async_copy(src_hbm.at[pl.ds(off, EDGE_CHUNK)], sbufs[s3],
                         isems_s[s3])
        pltpu.async_copy(dst_hbm.at[pl.ds(off, EDGE_CHUNK)], dbufs[s4],
                         isems_d[s4])

    def idx_wait(j, s3, s4):
        off = (c0 + j) * EDGE_CHUNK
        pltpu.make_async_copy(src_hbm.at[pl.ds(off, EDGE_CHUNK)], sbufs[s3],
                              isems_s[s3]).wait()
        pltpu.make_async_copy(dst_hbm.at[pl.ds(off, EDGE_CHUNK)], dbufs[s4],
                              isems_d[s4]).wait()

    def gather_start(s3, g2):
        pltpu.async_copy(z_hbm.at[sbufs[s3]], rows.at[s3], gsems[g2])

    def gather_wait(s3, g2):
        pltpu.make_async_copy(z_hbm.at[sbufs[s3]], rows.at[s3],
                              gsems[g2]).wait()

    def scat_start(s3, s4):
        pltpu.async_copy(rows.at[s3], acc.at[dbufs[s4]], ssems[s4], add=True)

    def scat_wait(s3, s4):
        pltpu.make_async_copy(rows.at[s3], acc.at[dbufs[s4]],
                              ssems[s4]).wait()

    idx_start(0, 0, 0)
    idx_start(1, 1, 1)

    cp = ((n // NS + 7) // 8) * 8
    tail = n - cp * (NS - 1)
    r0 = sid * cp

    def _init_range(start, length):
        @pl.when(cid == 0)
        def _():
            pltpu.sync_copy(z_hbm.at[pl.ds(start, length)],
                            acc.at[pl.ds(start, length)])

        @pl.when(cid == 1)
        def _():
            pltpu.sync_copy(zero_hbm.at[pl.ds(0, length)],
                            acc.at[pl.ds(start, length)])

    @pl.when(sid < NS - 1)
    def _():
        _init_range(r0, cp)

    @pl.when(sid == NS - 1)
    def _():
        _init_range(cp * (NS - 1), tail)

    idx_wait(0, 0, 0)
    gather_start(0, 0)
    plsc.subcore_barrier()

    def step(j, s3, s4, g2, scat_w, idx_st, nxt):
        if scat_w:
            scat_wait((s3 + 1) % 3, (s4 + 2) % 4)
        if nxt:
            idx_wait(j + 1, (s3 + 1) % 3, (s4 + 1) % 4)
            gather_start((s3 + 1) % 3, (g2 + 1) % 2)
        if idx_st:
            idx_start(j + 2, (s3 + 2) % 3, (s4 + 2) % 4)
        gather_wait(s3, g2)
        scat_start(s3, s4)

    for jj in range(4):
        step(jj, jj % 3, jj % 4, jj % 2,
             scat_w=(jj >= 2), idx_st=True, nxt=True)

    def body(k, carry):
        base = 4 + 12 * k
        for t in range(12):
            step(base + t, (4 + t) % 3, t % 4, t % 2,
                 scat_w=True, idx_st=True, nxt=True)
        return carry

    lax.fori_loop(0, (CPT - 8) // 12, body, 0)

    for jj in range(CPT - 4, CPT):
        step(jj, jj % 3, jj % 4, jj % 2,
             scat_w=True, idx_st=(jj + 2 < CPT), nxt=(jj + 1 < CPT))
    scat_wait((CPT - 2) % 3, (CPT - 2) % 4)
    scat_wait((CPT - 1) % 3, (CPT - 1) % 4)

    plsc.subcore_barrier()

    def _pub_range(start, length):
        pltpu.sync_copy(acc.at[pl.ds(start, length)],
                        out_hbm.at[pl.ds(cid * n + start, length)])

    @pl.when(sid < NS - 1)
    def _():
        _pub_range(r0, cp)

    @pl.when(sid == NS - 1)
    def _():
        _pub_range(cp * (NS - 1), tail)


def _make_scatter(n, d, dtype):
    mesh = plsc.VectorSubcoreMesh(core_axis_name="c", subcore_axis_name="s",
                                  num_cores=NC, num_subcores=NS)
    return pl.kernel(
        _scatter_body,
        out_type=jax.ShapeDtypeStruct((NC * n, d), dtype),
        mesh=mesh,
        scratch_types=(
            [pltpu.VMEM((EDGE_CHUNK,), jnp.int32) for _ in range(7)]
            + [pltpu.VMEM((NBUF, EDGE_CHUNK, d), dtype),
               pltpu.VMEM_SHARED((n + PAD_ROWS, d), dtype)]
            + [pltpu.SemaphoreType.DMA for _ in range(13)]
        ),
    )


def _mlp_body(p0_ref, p1_ref, wa_ref, ba_ref, wb_ref, bb_ref, o_ref):
    h = p0_ref[...] + p1_ref[...]
    h = jnp.maximum(
        jnp.dot(h, wa_ref[...], preferred_element_type=jnp.float32)
        + ba_ref[...], 0.0)
    h = jnp.dot(h, wb_ref[...], preferred_element_type=jnp.float32) + bb_ref[...]
    o_ref[...] = jnp.maximum(h, 0.0)


def _tail_body(p0_ref, p1_ref, wa_ref, ba_ref, wb_ref, bb_ref,
               bng_ref, bnb_ref, wp_ref, bp_ref, bn2g_ref, bn2b_ref, pa_ref,
               z_ref, p_ref):
    h = p0_ref[...] + p1_ref[...]
    h = jnp.maximum(
        jnp.dot(h, wa_ref[...], preferred_element_type=jnp.float32)
        + ba_ref[...], 0.0)
    h = jnp.dot(h, wb_ref[...], preferred_element_type=jnp.float32) + bb_ref[...]
    h = jnp.maximum(h, 0.0)
    z = h * (bng_ref[...] * BN_SCALE) + bnb_ref[...]
    z_ref[...] = z
    q = jnp.dot(z, wp_ref[...], preferred_element_type=jnp.float32) + bp_ref[...]
    q = q * (bn2g_ref[...] * BN_SCALE) + bn2b_ref[...]
    p_ref[...] = jnp.where(q > 0, q, pa_ref[0, 0] * q)


_BLK = 2000


def _row_spec(d):
    return pl.BlockSpec((_BLK, d), lambda i: (i, 0))


def _rep_spec(r, c):
    return pl.BlockSpec((r, c), lambda i: (0, 0))


def _half_spec(d, half_blocks):
    return pl.BlockSpec((_BLK, d), lambda i, h=half_blocks: (i + h, 0))


def _mlp_call(agg, wa, ba, wb, bb):
    n = agg.shape[0] // 2
    d = agg.shape[1]
    h = wa.shape[1]
    nb = n // _BLK
    return pl.pallas_call(
        _mlp_body,
        grid=(nb,),
        in_specs=[_half_spec(d, 0), _half_spec(d, nb),
                  _rep_spec(d, h), _rep_spec(1, h),
                  _rep_spec(h, h), _rep_spec(1, h)],
        out_specs=_row_spec(h),
        out_shape=jax.ShapeDtypeStruct((n, h), agg.dtype),
    )(agg, agg, wa, ba.reshape(1, h), wb, bb.reshape(1, h))


def _tail_call(agg, wa, ba, wb, bb, bng, bnb, wp, bp, bn2g, bn2b, pa):
    n = agg.shape[0] // 2
    d = agg.shape[1]
    h = wa.shape[1]
    nb = n // _BLK
    vecs = [a.reshape(1, h) for a in (ba, bb, bng, bnb, bp, bn2g, bn2b)]
    return pl.pallas_call(
        _tail_body,
        grid=(nb,),
        in_specs=[_half_spec(d, 0), _half_spec(d, nb),
                  _rep_spec(d, h), _rep_spec(1, h),
                  _rep_spec(h, h), _rep_spec(1, h),
                  _rep_spec(1, h), _rep_spec(1, h),
                  _rep_spec(h, h), _rep_spec(1, h),
                  _rep_spec(1, h), _rep_spec(1, h),
                  pl.BlockSpec(memory_space=pltpu.SMEM)],
        out_specs=[_row_spec(h), _row_spec(h)],
        out_shape=[jax.ShapeDtypeStruct((n, h), agg.dtype),
                   jax.ShapeDtypeStruct((n, h), agg.dtype)],
    )(agg, agg, wa, vecs[0], wb, vecs[1], vecs[2], vecs[3], wp, vecs[4],
      vecs[5], vecs[6], pa.reshape(1, 1))


def kernel(x, edge_index, W1a, b1a, W1b, b1b, W2a, b2a, W2b, b2b,
           bn_gamma, bn_beta, Wp, bp, bn2_gamma, bn2_beta, prelu_a):
    n, d = x.shape
    e = edge_index.shape[1]
    e_pad = NW * CPT * EDGE_CHUNK
    ar = jnp.arange(e_pad - e, dtype=jnp.int32)
    src = jnp.concatenate([edge_index[0], ar % n])
    dst = jnp.concatenate([edge_index[1], n + (ar % PAD_ROWS)])
    cp = ((n // NS + 7) // 8) * 8
    zeros = jnp.zeros((cp, d), x.dtype)
    scat = _make_scatter(n, d, x.dtype)

    agg1 = scat(x, src, dst, zeros)
    z1 = _mlp_call(agg1, W1a, b1a, W1b, b1b)
    agg2 = scat(z1, src, dst, zeros)
    z, p = _tail_call(agg2, W2a, b2a, W2b, b2b,
                      bn_gamma, bn_beta, Wp, bp, bn2_gamma, bn2_beta, prelu_a)
    return (z, p)
